# Initial kernel scaffold; baseline (speedup 1.0000x reference)
import jax
import jax.numpy as jnp
from jax.experimental import pallas as pl


def kernel(boxes, scores):
    raise NotImplementedError("write your pallas kernel here")


# speedup vs baseline: 5.7115x; 5.7115x over previous
import functools

import jax
import jax.numpy as jnp
from jax import lax
from jax.experimental import pallas as pl
from jax.experimental.pallas import tpu as pltpu
from jax.experimental.pallas import tpu_sc as plsc

_N = 20000
_IOU_THR = 0.5
_SCORE_THR = 0.1
_MAX_KEEP = 256
_NEG = -1e10

_L = 16
_NS = 16
_NPAD = 20480
_SHARD = _NPAD // _NS
_NSLICE = _SHARD // _L
_BIGI = 2**31 - 1
_FNEG = -3.0e38


def _nms_body(x1_h, y1_h, x2_h, y2_h, s_h, out_h,
              x1_v, y1_v, x2_v, y2_v, s_v, area_v, kept_v, tab_v, comm_v, tbl_sh):
    wid = lax.axis_index("s")
    loff = wid * _SHARD
    iota = lax.iota(jnp.int32, _L)
    zeros_i = jnp.zeros((_L,), jnp.int32)
    ones_i = jnp.full((_L,), 1, jnp.int32)

    pltpu.sync_copy(x1_h, x1_v)
    pltpu.sync_copy(y1_h, y1_v)
    pltpu.sync_copy(x2_h, x2_v)
    pltpu.sync_copy(y2_h, y2_v)
    pltpu.sync_copy(s_h.at[pl.ds(loff, _SHARD)], s_v)

    def _argmax_allreduce(v, idx):
        for sh in (8, 4, 2, 1):
            perm = iota ^ sh
            v2 = v.at[perm].get(mode="promise_in_bounds")
            i2 = idx.at[perm].get(mode="promise_in_bounds")
            take = (v2 > v) | ((v2 == v) & (i2 < idx))
            v = jnp.where(take, v2, v)
            idx = jnp.where(take, i2, idx)
        return v, idx

    def _publish(curmax, curidx, slot):
        vb, ib = _argmax_allreduce(curmax, curidx)
        row = jnp.where(iota == 0, vb, plsc.bitcast(ib, jnp.float32))
        comm_v[...] = row
        pltpu.sync_copy(comm_v, tbl_sh.at[slot, wid])
        plsc.subcore_barrier()

    def _pro(i, carry):
        curmax, curidx = carry
        sl = pl.ds(i * _L, _L)
        gsl = pl.ds(loff + i * _L, _L)
        v = s_v[sl]
        v = jnp.where(v > _SCORE_THR, v, _NEG)
        s_v[sl] = v
        area_v[sl] = (jnp.maximum(x2_v[gsl] - x1_v[gsl], 0.0)
                      * jnp.maximum(y2_v[gsl] - y1_v[gsl], 0.0))
        upd = v > curmax
        curmax = jnp.where(upd, v, curmax)
        curidx = jnp.where(upd, loff + i * _L + iota, curidx)
        return curmax, curidx

    cm0 = jnp.full((_L,), _FNEG, jnp.float32)
    cm, ci = lax.fori_loop(0, _NSLICE, _pro, (cm0, zeros_i))
    _publish(cm, ci, 0)

    def _round(t, _):
        pltpu.sync_copy(tbl_sh.at[t % 2], tab_v)
        vals = plsc.load_gather(tab_v, [iota, zeros_i])
        gidx = plsc.bitcast(plsc.load_gather(tab_v, [iota, ones_i]), jnp.int32)
        mb, widx_v = _argmax_allreduce(vals, gidx)
        validv = mb > (_NEG / 2.0)

        bx1 = plsc.load_gather(x1_v, [widx_v])
        by1 = plsc.load_gather(y1_v, [widx_v])
        bx2 = plsc.load_gather(x2_v, [widx_v])
        by2 = plsc.load_gather(y2_v, [widx_v])
        a1 = (jnp.maximum(bx2 - bx1, 0.0) * jnp.maximum(by2 - by1, 0.0))

        zf = jnp.zeros((_L,), jnp.float32)
        kept_v[t, :] = jnp.where(validv, bx1, zf)
        kept_v[t + _MAX_KEEP, :] = jnp.where(validv, by1, zf)
        kept_v[t + 2 * _MAX_KEEP, :] = jnp.where(validv, bx2, zf)
        kept_v[t + 3 * _MAX_KEEP, :] = jnp.where(validv, by2, zf)
        kept_v[t + 4 * _MAX_KEEP, :] = jnp.where(validv, mb, zf)

        def _pass(i, carry):
            curmax, curidx = carry
            sl = pl.ds(i * _L, _L)
            gsl = pl.ds(loff + i * _L, _L)
            idxv = loff + i * _L + iota
            v = s_v[sl]
            iw = jnp.maximum(jnp.minimum(bx2, x2_v[gsl])
                             - jnp.maximum(bx1, x1_v[gsl]), 0.0)
            ih = jnp.maximum(jnp.minimum(by2, y2_v[gsl])
                             - jnp.maximum(by1, y1_v[gsl]), 0.0)
            inter = iw * ih
            iou = inter / (a1 + area_v[sl] - inter + 1e-6)
            supp = ((iou > _IOU_THR) | (idxv == widx_v)) & validv
            vn = jnp.where(supp, _NEG, v)
            s_v[sl] = vn
            upd = vn > curmax
            curmax = jnp.where(upd, vn, curmax)
            curidx = jnp.where(upd, idxv, curidx)
            return curmax, curidx

        cm, ci = lax.fori_loop(0, _NSLICE, _pass,
                               (jnp.full((_L,), _FNEG, jnp.float32), zeros_i))
        _publish(cm, ci, (t + 1) % 2)
        return 0

    lax.fori_loop(0, _MAX_KEEP, _round, 0)

    @pl.when(wid == 0)
    def _():
        pltpu.sync_copy(kept_v, out_h)


@jax.jit
def _nms_sc(x1, y1, x2, y2, s):
    mesh = plsc.VectorSubcoreMesh(core_axis_name="c", subcore_axis_name="s",
                                  num_cores=1)
    f = pl.kernel(
        _nms_body,
        out_type=jax.ShapeDtypeStruct((5 * _MAX_KEEP, _L), jnp.float32),
        mesh=mesh,
        compiler_params=pltpu.CompilerParams(needs_layout_passes=False,
                                             use_tc_tiling_on_sc=False),
        scratch_types=[
            pltpu.VMEM((_NPAD,), jnp.float32),
            pltpu.VMEM((_NPAD,), jnp.float32),
            pltpu.VMEM((_NPAD,), jnp.float32),
            pltpu.VMEM((_NPAD,), jnp.float32),
            pltpu.VMEM((_SHARD,), jnp.float32),
            pltpu.VMEM((_SHARD,), jnp.float32),
            pltpu.VMEM((5 * _MAX_KEEP, _L), jnp.float32),
            pltpu.VMEM((_NS, _L), jnp.float32),
            pltpu.VMEM((_L,), jnp.float32),
            pltpu.VMEM_SHARED((2, _NS, _L), jnp.float32),
        ],
    )
    return f(x1, y1, x2, y2, s)


def kernel(boxes, scores):
    pad = _NPAD - _N
    x1 = jnp.pad(boxes[:, 0], (0, pad))
    y1 = jnp.pad(boxes[:, 1], (0, pad))
    x2 = jnp.pad(boxes[:, 2], (0, pad))
    y2 = jnp.pad(boxes[:, 3], (0, pad))
    s = jnp.pad(scores, (0, pad))
    out = _nms_sc(x1, y1, x2, y2, s)
    return out[:, 0].reshape(5, _MAX_KEEP).T


# speedup vs baseline: 15.5156x; 2.7165x over previous
import functools

import jax
import jax.numpy as jnp
from jax import lax
from jax.experimental import pallas as pl
from jax.experimental.pallas import tpu as pltpu
from jax.experimental.pallas import tpu_sc as plsc

_N = 20000
_IOU_THR = 0.5
_SCORE_THR = 0.1
_MAX_KEEP = 256
_NEG = -1e10

_L = 16
_NS = 16
_NPAD = 20480
_SHARD = _NPAD // _NS
_NSLICE = _SHARD // _L
_BIGI = 2**31 - 1
_FNEG = -3.0e38


def _nms_body(x1_h, y1_h, x2_h, y2_h, s_h, out_h,
              x1_v, y1_v, x2_v, y2_v, s_v, area_v, kept_v, tab_v, comm_v, tbl_sh):
    wid = lax.axis_index("s")
    loff = wid * _SHARD
    iota = lax.iota(jnp.int32, _L)
    zeros_i = jnp.zeros((_L,), jnp.int32)
    ones_i = jnp.full((_L,), 1, jnp.int32)

    pltpu.sync_copy(x1_h, x1_v)
    pltpu.sync_copy(y1_h, y1_v)
    pltpu.sync_copy(x2_h, x2_v)
    pltpu.sync_copy(y2_h, y2_v)
    pltpu.sync_copy(s_h.at[pl.ds(loff, _SHARD)], s_v)

    def _argmax_allreduce(v, idx):
        for sh in (8, 4, 2, 1):
            perm = iota ^ sh
            v2 = v.at[perm].get(mode="promise_in_bounds")
            i2 = idx.at[perm].get(mode="promise_in_bounds")
            take = (v2 > v) | ((v2 == v) & (i2 < idx))
            v = jnp.where(take, v2, v)
            idx = jnp.where(take, i2, idx)
        return v, idx

    def _publish(curmax, curidx, slot):
        vb, ib = _argmax_allreduce(curmax, curidx)
        row = jnp.where(iota == 0, vb, plsc.bitcast(ib, jnp.float32))
        comm_v[...] = row
        pltpu.sync_copy(comm_v, tbl_sh.at[slot, wid])
        plsc.subcore_barrier()

    cm0 = jnp.full((_L,), _FNEG, jnp.float32)

    @plsc.parallel_loop(0, _NSLICE, unroll=8, carry=(cm0, zeros_i))
    def _pro(i, carry):
        curmax, curidx = carry
        sl = pl.ds(i * _L, _L)
        gsl = pl.ds(loff + i * _L, _L)
        v = s_v[sl]
        v = jnp.where(v > _SCORE_THR, v, _NEG)
        s_v[sl] = v
        area_v[sl] = (jnp.maximum(x2_v[gsl] - x1_v[gsl], 0.0)
                      * jnp.maximum(y2_v[gsl] - y1_v[gsl], 0.0))
        upd = v > curmax
        curmax = jnp.where(upd, v, curmax)
        curidx = jnp.where(upd, loff + i * _L + iota, curidx)
        return curmax, curidx

    cm, ci = _pro
    _publish(cm, ci, 0)

    def _round(t, _):
        pltpu.sync_copy(tbl_sh.at[t % 2], tab_v)
        vals = plsc.load_gather(tab_v, [iota, zeros_i])
        gidx = plsc.bitcast(plsc.load_gather(tab_v, [iota, ones_i]), jnp.int32)
        mb, widx_v = _argmax_allreduce(vals, gidx)
        validv = mb > (_NEG / 2.0)

        bx1 = plsc.load_gather(x1_v, [widx_v])
        by1 = plsc.load_gather(y1_v, [widx_v])
        bx2 = plsc.load_gather(x2_v, [widx_v])
        by2 = plsc.load_gather(y2_v, [widx_v])
        a1 = (jnp.maximum(bx2 - bx1, 0.0) * jnp.maximum(by2 - by1, 0.0))

        zf = jnp.zeros((_L,), jnp.float32)
        kept_v[t, :] = jnp.where(validv, bx1, zf)
        kept_v[t + _MAX_KEEP, :] = jnp.where(validv, by1, zf)
        kept_v[t + 2 * _MAX_KEEP, :] = jnp.where(validv, bx2, zf)
        kept_v[t + 3 * _MAX_KEEP, :] = jnp.where(validv, by2, zf)
        kept_v[t + 4 * _MAX_KEEP, :] = jnp.where(validv, mb, zf)

        @plsc.parallel_loop(0, _NSLICE, unroll=8,
                            carry=(jnp.full((_L,), _FNEG, jnp.float32), zeros_i))
        def _pass(i, carry):
            curmax, curidx = carry
            sl = pl.ds(i * _L, _L)
            gsl = pl.ds(loff + i * _L, _L)
            idxv = loff + i * _L + iota
            v = s_v[sl]
            iw = jnp.maximum(jnp.minimum(bx2, x2_v[gsl])
                             - jnp.maximum(bx1, x1_v[gsl]), 0.0)
            ih = jnp.maximum(jnp.minimum(by2, y2_v[gsl])
                             - jnp.maximum(by1, y1_v[gsl]), 0.0)
            inter = iw * ih
            iou = inter / (a1 + area_v[sl] - inter + 1e-6)
            supp = ((iou > _IOU_THR) | (idxv == widx_v)) & validv
            vn = jnp.where(supp, _NEG, v)
            s_v[sl] = vn
            upd = vn > curmax
            curmax = jnp.where(upd, vn, curmax)
            curidx = jnp.where(upd, idxv, curidx)
            return curmax, curidx

        cm, ci = _pass
        _publish(cm, ci, (t + 1) % 2)
        return 0

    lax.fori_loop(0, _MAX_KEEP, _round, 0)

    @pl.when(wid == 0)
    def _():
        pltpu.sync_copy(kept_v, out_h)


@jax.jit
def _nms_sc(x1, y1, x2, y2, s):
    mesh = plsc.VectorSubcoreMesh(core_axis_name="c", subcore_axis_name="s",
                                  num_cores=1)
    f = pl.kernel(
        _nms_body,
        out_type=jax.ShapeDtypeStruct((5 * _MAX_KEEP, _L), jnp.float32),
        mesh=mesh,
        compiler_params=pltpu.CompilerParams(needs_layout_passes=False,
                                             use_tc_tiling_on_sc=False),
        scratch_types=[
            pltpu.VMEM((_NPAD,), jnp.float32),
            pltpu.VMEM((_NPAD,), jnp.float32),
            pltpu.VMEM((_NPAD,), jnp.float32),
            pltpu.VMEM((_NPAD,), jnp.float32),
            pltpu.VMEM((_SHARD,), jnp.float32),
            pltpu.VMEM((_SHARD,), jnp.float32),
            pltpu.VMEM((5 * _MAX_KEEP, _L), jnp.float32),
            pltpu.VMEM((_NS, _L), jnp.float32),
            pltpu.VMEM((_L,), jnp.float32),
            pltpu.VMEM_SHARED((2, _NS, _L), jnp.float32),
        ],
    )
    return f(x1, y1, x2, y2, s)


def kernel(boxes, scores):
    pad = _NPAD - _N
    x1 = jnp.pad(boxes[:, 0], (0, pad))
    y1 = jnp.pad(boxes[:, 1], (0, pad))
    x2 = jnp.pad(boxes[:, 2], (0, pad))
    y2 = jnp.pad(boxes[:, 3], (0, pad))
    s = jnp.pad(scores, (0, pad))
    out = _nms_sc(x1, y1, x2, y2, s)
    return out[:, 0].reshape(5, _MAX_KEEP).T
